# baseline (speedup 1.0000x reference)
import jax
import jax.numpy as jnp
from jax.experimental import pallas as pl
from jax.experimental.pallas import tpu as pltpu

_NSTREAMS = 32


def kernel(inputs, table):
    del inputs
    rows, dim = table.shape
    slab = rows // _NSTREAMS

    def body(in_ref, out_ref, sem):
        copies = [
            pltpu.make_async_copy(
                in_ref.at[pl.ds(j * slab, slab), :],
                out_ref.at[pl.ds(j * slab, slab), :],
                sem,
            )
            for j in range(_NSTREAMS)
        ]
        for c in copies:
            c.start()
        for c in copies:
            c.wait()

    return pl.pallas_call(
        body,
        in_specs=[pl.BlockSpec(memory_space=pl.ANY)],
        out_specs=pl.BlockSpec(memory_space=pl.ANY),
        scratch_shapes=[pltpu.SemaphoreType.DMA],
        out_shape=jax.ShapeDtypeStruct(table.shape, table.dtype),
    )(table)


# speedup vs baseline: 42.1798x; 42.1798x over previous
import jax
import jax.numpy as jnp
from jax.experimental import pallas as pl
from jax.experimental.pallas import tpu as pltpu

_CHUNK = 512
_NBUF = 4


def kernel(inputs, table):
    del inputs
    rows, dim = table.shape
    nchunks = rows // _CHUNK

    def body(in_ref, out_ref, buf, rsem, wsem):
        def read(c):
            return pltpu.make_async_copy(
                in_ref.at[pl.ds(c * _CHUNK, _CHUNK), :],
                buf.at[c % _NBUF],
                rsem,
            )

        def write(c):
            return pltpu.make_async_copy(
                buf.at[c % _NBUF],
                out_ref.at[pl.ds(c * _CHUNK, _CHUNK), :],
                wsem,
            )

        reads = [None] * nchunks
        writes = [None] * nchunks
        for c in range(min(_NBUF, nchunks)):
            reads[c] = read(c)
            reads[c].start()
        for c in range(nchunks):
            reads[c].wait()
            writes[c] = write(c)
            writes[c].start()
            nxt = c + _NBUF
            if nxt < nchunks:
                writes[c].wait()
                reads[nxt] = read(nxt)
                reads[nxt].start()
        for c in range(max(0, nchunks - _NBUF), nchunks):
            writes[c].wait()

    return pl.pallas_call(
        body,
        in_specs=[pl.BlockSpec(memory_space=pl.ANY)],
        out_specs=pl.BlockSpec(memory_space=pl.ANY),
        scratch_shapes=[
            pltpu.VMEM((_NBUF, _CHUNK, dim), table.dtype),
            pltpu.SemaphoreType.DMA,
            pltpu.SemaphoreType.DMA,
        ],
        out_shape=jax.ShapeDtypeStruct(table.shape, table.dtype),
    )(table)


# speedup vs baseline: 48.4454x; 1.1485x over previous
import jax
import jax.numpy as jnp
from jax.experimental import pallas as pl
from jax.experimental.pallas import tpu as pltpu

_CHUNK = 1024
_NBUF = 4


def kernel(inputs, table):
    del inputs
    rows, dim = table.shape
    nchunks = rows // _CHUNK

    def body(in_ref, out_ref, buf, rsem, wsem):
        def read(c):
            return pltpu.make_async_copy(
                in_ref.at[pl.ds(c * _CHUNK, _CHUNK), :],
                buf.at[c % _NBUF],
                rsem,
            )

        def write(c):
            return pltpu.make_async_copy(
                buf.at[c % _NBUF],
                out_ref.at[pl.ds(c * _CHUNK, _CHUNK), :],
                wsem,
            )

        reads = [None] * nchunks
        writes = [None] * nchunks
        for c in range(min(_NBUF, nchunks)):
            reads[c] = read(c)
            reads[c].start()
        for c in range(nchunks):
            reads[c].wait()
            writes[c] = write(c)
            writes[c].start()
            nxt = c + _NBUF
            if nxt < nchunks:
                writes[c].wait()
                reads[nxt] = read(nxt)
                reads[nxt].start()
        for c in range(max(0, nchunks - _NBUF), nchunks):
            writes[c].wait()

    return pl.pallas_call(
        body,
        in_specs=[pl.BlockSpec(memory_space=pl.ANY)],
        out_specs=pl.BlockSpec(memory_space=pl.ANY),
        scratch_shapes=[
            pltpu.VMEM((_NBUF, _CHUNK, dim), table.dtype),
            pltpu.SemaphoreType.DMA,
            pltpu.SemaphoreType.DMA,
        ],
        out_shape=jax.ShapeDtypeStruct(table.shape, table.dtype),
    )(table)
